# baseline (speedup 1.0000x reference)
import functools

import jax
import jax.numpy as jnp
from jax import lax
from jax.experimental import pallas as pl
from jax.experimental.pallas import tpu as pltpu
from jax.experimental.pallas import tpu_sc as plsc

D_EMBD = 32
N_TAB = 1000000
B_TOTAL = 16384 * 50

_info = plsc.get_sparse_core_info()
_NC = _info.num_cores
_NS = _info.num_subcores
_NW = _NC * _NS
_B_PER_W = B_TOTAL // _NW
_C = 320
_NCH = _B_PER_W // _C
_G = _C // 16

_mesh = plsc.VectorSubcoreMesh(core_axis_name="c", subcore_axis_name="s")


@functools.partial(
    pl.kernel,
    mesh=_mesh,
    out_type=jax.ShapeDtypeStruct((B_TOTAL // 4, 128), jnp.float32),
    scratch_types=[
        pltpu.VMEM((_C,), jnp.int32),
        pltpu.VMEM((_C,), jnp.int32),
        pltpu.VMEM((_C,), jnp.int32),
        pltpu.VMEM((_C,), jnp.int32),
        pltpu.VMEM((_C, 128), jnp.float32),
        pltpu.VMEM((_C, 128), jnp.float32),
        pltpu.VMEM((_C // 4, 128), jnp.float32),
        pltpu.VMEM((_C // 4, 128), jnp.float32),
        pltpu.SemaphoreType.DMA,
        pltpu.SemaphoreType.DMA,
        pltpu.SemaphoreType.DMA,
        pltpu.SemaphoreType.DMA,
        pltpu.SemaphoreType.DMA,
        pltpu.SemaphoreType.DMA,
        pltpu.SemaphoreType.DMA,
        pltpu.SemaphoreType.DMA,
    ],
    compiler_params=pltpu.CompilerParams(needs_layout_passes=False),
)
def _gather(q_hbm, rem_hbm, tab4_hbm, out_hbm,
            qv0, qv1, rv0, rv1, sv0, sv1, ov0, ov1,
            sq0, sq1, sr0, sr1, sg0, sg1, so0, so1):
    wid = lax.axis_index("s") * _NC + lax.axis_index("c")
    base = wid * _B_PER_W
    obase = base // 4
    qv = (qv0, qv1)
    rv = (rv0, rv1)
    sv = (sv0, sv1)
    ov = (ov0, ov1)
    sq = (sq0, sq1)
    sr = (sr0, sr1)
    sg = (sg0, sg1)
    so = (so0, so1)
    iota16 = lax.iota(jnp.int32, 16)

    def q_slice(j):
        return pl.ds(pl.multiple_of(base + j * _C, 8), _C)

    def o_slice(j):
        return pl.ds(pl.multiple_of(obase + j * (_C // 4), 8), _C // 4)

    def start_qrem(j, b):
        pltpu.async_copy(q_hbm.at[q_slice(j)], qv[b], sq[b])
        pltpu.async_copy(rem_hbm.at[q_slice(j)], rv[b], sr[b])

    def wait_qrem(j, b):
        pltpu.make_async_copy(q_hbm.at[q_slice(j)], qv[b], sq[b]).wait()
        pltpu.make_async_copy(rem_hbm.at[q_slice(j)], rv[b], sr[b]).wait()

    def start_gather(b):
        pltpu.async_copy(tab4_hbm.at[qv[b]], sv[b], sg[b])

    def wait_gather(b):
        pltpu.make_async_copy(tab4_hbm.at[qv[b]], sv[b], sg[b]).wait()

    def start_out(j, b):
        pltpu.async_copy(ov[b], out_hbm.at[o_slice(j)], so[b])

    def wait_out(j, b):
        pltpu.make_async_copy(ov[b], out_hbm.at[o_slice(j)], so[b]).wait()

    def extract(b):
        def grp_body(g, _):
            rows16 = g * 16 + iota16
            rem16 = rv[b][pl.ds(g * 16, 16)]
            colv = rem16 * 32
            dstrow16 = lax.shift_right_logical(rows16, 2)
            dstcolv = (rows16 & 3) * 32
            for jc in range(D_EMBD):
                v = plsc.load_gather(sv[b], [rows16, colv])
                plsc.store_scatter(ov[b], [dstrow16, dstcolv], v)
                colv = colv + 1
                dstcolv = dstcolv + 1
            return 0
        lax.fori_loop(0, _G, grp_body, 0, unroll=False)

    start_qrem(0, 0)
    start_qrem(1, 1)
    wait_qrem(0, 0)
    start_gather(0)

    def chunk_body(jj, _):
        for b in range(2):
            j = jj * 2 + b

            @pl.when(j + 1 < _NCH)
            def _():
                wait_qrem(j + 1, 1 - b)
                start_gather(1 - b)

            wait_gather(b)

            @pl.when(j >= 2)
            def _():
                wait_out(j - 2, b)

            extract(b)
            start_out(j, b)

            @pl.when(j + 2 < _NCH)
            def _():
                start_qrem(j + 2, b)
        return 0

    lax.fori_loop(0, _NCH // 2, chunk_body, 0, unroll=False)
    wait_out(_NCH - 2, 0)
    wait_out(_NCH - 1, 1)


def kernel(x, embd):
    flat_idx = x.reshape(-1).astype(jnp.int32)
    q = flat_idx >> 2
    rem = flat_idx & 3
    tab4 = embd.reshape(N_TAB // 4, 128)
    out4 = _gather(q, rem, tab4)
    return out4.reshape(x.shape[0], x.shape[1], D_EMBD)


# speedup vs baseline: 1.0831x; 1.0831x over previous
import functools

import jax
import jax.numpy as jnp
from jax import lax
from jax.experimental import pallas as pl
from jax.experimental.pallas import tpu as pltpu
from jax.experimental.pallas import tpu_sc as plsc

D_EMBD = 32
N_TAB = 1000000
B_TOTAL = 16384 * 50

_info = plsc.get_sparse_core_info()
_NC = _info.num_cores
_NS = _info.num_subcores
_NW = _NC * _NS
_B_PER_W = B_TOTAL // _NW
_C = 320
_NCH = _B_PER_W // _C
_G = _C // 16

_mesh = plsc.VectorSubcoreMesh(core_axis_name="c", subcore_axis_name="s")


@functools.partial(
    pl.kernel,
    mesh=_mesh,
    out_type=jax.ShapeDtypeStruct((B_TOTAL // 4, 128), jnp.float32),
    scratch_types=[
        pltpu.VMEM((_C,), jnp.int32),
        pltpu.VMEM((_C,), jnp.int32),
        pltpu.VMEM((_C,), jnp.int32),
        pltpu.VMEM((_C,), jnp.int32),
        pltpu.VMEM((_C, 128), jnp.float32),
        pltpu.VMEM((_C, 128), jnp.float32),
        pltpu.VMEM((_C // 4, 128), jnp.float32),
        pltpu.VMEM((_C // 4, 128), jnp.float32),
        pltpu.SemaphoreType.DMA,
        pltpu.SemaphoreType.DMA,
        pltpu.SemaphoreType.DMA,
        pltpu.SemaphoreType.DMA,
        pltpu.SemaphoreType.DMA,
        pltpu.SemaphoreType.DMA,
        pltpu.SemaphoreType.DMA,
        pltpu.SemaphoreType.DMA,
    ],
    compiler_params=pltpu.CompilerParams(needs_layout_passes=False),
)
def _gather(q_hbm, rem_hbm, tab4_hbm, out_hbm,
            qv0, qv1, rv0, rv1, sv0, sv1, ov0, ov1,
            sq0, sq1, sr0, sr1, sg0, sg1, so0, so1):
    wid = lax.axis_index("s") * _NC + lax.axis_index("c")
    base = wid * _B_PER_W
    qv = (qv0, qv1)
    rv = (rv0, rv1)
    sv = (sv0, sv1)
    ov = (ov0, ov1)
    sq = (sq0, sq1)
    sr = (sr0, sr1)
    sg = (sg0, sg1)
    so = (so0, so1)
    iota16 = lax.iota(jnp.int32, 16)

    obase = base // 4

    def idx_slice(j):
        return pl.ds(pl.multiple_of(base + j * _C, 8), _C)

    def out_slice(j):
        return pl.ds(pl.multiple_of(obase + j * (_C // 4), 8), _C // 4)

    def start_qrem(j, b):
        pltpu.async_copy(q_hbm.at[idx_slice(j)], qv[b], sq[b])
        pltpu.async_copy(rem_hbm.at[idx_slice(j)], rv[b], sr[b])

    def wait_qrem(j, b):
        pltpu.make_async_copy(q_hbm.at[idx_slice(j)], qv[b], sq[b]).wait()
        pltpu.make_async_copy(rem_hbm.at[idx_slice(j)], rv[b], sr[b]).wait()

    def start_gather(b):
        pltpu.async_copy(tab4_hbm.at[qv[b]], sv[b], sg[b])

    def wait_gather(b):
        pltpu.make_async_copy(tab4_hbm.at[qv[b]], sv[b], sg[b]).wait()

    def start_out(j, b):
        pltpu.async_copy(ov[b], out_hbm.at[out_slice(j)], so[b])

    def wait_out(j, b):
        pltpu.make_async_copy(ov[b], out_hbm.at[out_slice(j)], so[b]).wait()

    def extract(b):
        @plsc.parallel_loop(0, _G, unroll=2)
        def _grp(g):
            rows16 = g * 16 + iota16
            rem16 = rv[b][pl.ds(pl.multiple_of(g * 16, 16), 16)]
            src = rem16 * 32
            dstrow16 = lax.shift_right_logical(rows16, 2)
            dst = (rows16 & 3) * 32
            for _jc in range(D_EMBD):
                v = plsc.load_gather(sv[b], [rows16, src])
                plsc.store_scatter(ov[b], [dstrow16, dst], v)
                src = src + 1
                dst = dst + 1

    start_qrem(0, 0)
    start_qrem(1, 1)
    wait_qrem(0, 0)
    start_gather(0)

    def chunk_body(jj, _):
        for b in range(2):
            j = jj * 2 + b

            wait_gather(b)

            @pl.when(j + 1 < _NCH)
            def _():
                wait_qrem(j + 1, 1 - b)
                start_gather(1 - b)

            @pl.when(j >= 2)
            def _():
                wait_out(j - 2, b)

            extract(b)
            start_out(j, b)

            @pl.when(j + 2 < _NCH)
            def _():
                start_qrem(j + 2, b)
        return 0

    lax.fori_loop(0, _NCH // 2, chunk_body, 0, unroll=False)
    wait_out(_NCH - 2, 0)
    wait_out(_NCH - 1, 1)


def kernel(x, embd):
    flat_idx = x.reshape(-1).astype(jnp.int32)
    q = flat_idx >> 2
    rem = flat_idx & 3
    tab4 = embd.reshape(N_TAB // 4, 128)
    out4 = _gather(q, rem, tab4)
    out3 = out4.reshape(x.shape[0], x.shape[1], D_EMBD)
    return jnp.where(out3 == out3, out3, jnp.float32(0))


# speedup vs baseline: 1.1746x; 1.0845x over previous
import functools

import jax
import jax.numpy as jnp
from jax import lax
from jax.experimental import pallas as pl
from jax.experimental.pallas import tpu as pltpu
from jax.experimental.pallas import tpu_sc as plsc

D_EMBD = 32
B_TOTAL = 16384 * 50

_info = plsc.get_sparse_core_info()
_NC = _info.num_cores
_NS = _info.num_subcores
_NW = _NC * _NS
_B_PER_W = B_TOTAL // _NW
_CHUNK = 1600
_N_CHUNKS = _B_PER_W // _CHUNK

_mesh = plsc.VectorSubcoreMesh(core_axis_name="c", subcore_axis_name="s")


@functools.partial(
    pl.kernel,
    mesh=_mesh,
    out_type=jax.ShapeDtypeStruct((B_TOTAL, D_EMBD), jnp.float32),
    scratch_types=[
        pltpu.VMEM((_CHUNK,), jnp.int32),
        pltpu.VMEM((_CHUNK,), jnp.int32),
        pltpu.VMEM((_CHUNK, D_EMBD), jnp.float32),
        pltpu.VMEM((_CHUNK, D_EMBD), jnp.float32),
        pltpu.SemaphoreType.DMA,
        pltpu.SemaphoreType.DMA,
        pltpu.SemaphoreType.DMA,
        pltpu.SemaphoreType.DMA,
        pltpu.SemaphoreType.DMA,
        pltpu.SemaphoreType.DMA,
    ],
    compiler_params=pltpu.CompilerParams(use_tc_tiling_on_sc=False),
)
def _gather(idx_hbm, tab_hbm, out_hbm,
            idx_v0, idx_v1, rows_v0, rows_v1,
            si0, si1, sg0, sg1, so0, so1):
    wid = lax.axis_index("s") * _NC + lax.axis_index("c")
    base = wid * _B_PER_W
    idx_v = (idx_v0, idx_v1)
    rows_v = (rows_v0, rows_v1)
    sem_i = (si0, si1)
    sem_g = (sg0, sg1)
    sem_o = (so0, so1)

    def idx_off(j):
        return pl.ds(base + j * _CHUNK, _CHUNK)

    ih = [None, None]
    gh = [None, None]
    oh = [None, None]
    ih[0] = pltpu.async_copy(idx_hbm.at[idx_off(0)], idx_v[0], sem_i[0])
    ih[1] = pltpu.async_copy(idx_hbm.at[idx_off(1)], idx_v[1], sem_i[1])
    for j in range(_N_CHUNKS):
        b = j & 1
        if oh[b] is not None:
            oh[b].wait()
        ih[b].wait()
        gh[b] = pltpu.async_copy(tab_hbm.at[idx_v[b]], rows_v[b], sem_g[b])
        if j >= 1:
            gh[1 - b].wait()
            oh[1 - b] = pltpu.async_copy(
                rows_v[1 - b], out_hbm.at[idx_off(j - 1)], sem_o[1 - b])
            if j + 1 < _N_CHUNKS:
                ih[1 - b] = pltpu.async_copy(
                    idx_hbm.at[idx_off(j + 1)], idx_v[1 - b], sem_i[1 - b])
    bl = (_N_CHUNKS - 1) & 1
    gh[bl].wait()
    oh[bl] = pltpu.async_copy(
        rows_v[bl], out_hbm.at[idx_off(_N_CHUNKS - 1)], sem_o[bl])
    oh[1 - bl].wait()
    oh[bl].wait()


def kernel(x, embd):
    flat_idx = x.reshape(-1).astype(jnp.int32)
    out = _gather(flat_idx, embd)
    return out.reshape(x.shape[0], x.shape[1], D_EMBD)
